# baseline (speedup 1.0000x reference)
import numpy as np
import jax
import jax.numpy as jnp
from jax.experimental import pallas as pl
from jax.experimental.pallas import tpu as pltpu

B = 32
V = 100000
NSTEP = 32
KDRAFT = 4
LDRAFT = 8
W = 2048
VPAD = 106496
NCHUNK = VPAD // W

TINY = np.float32(np.finfo(np.float32).tiny)
UMUL = np.float32(np.float32(1.0) - TINY)
NEG_INF = np.float32(-np.inf)
BIG_COL = np.int32(2**30)


def _threefry2x32_host(k1, k2, x0, x1):
    u32 = np.uint32
    rots = ([13, 15, 26, 6], [17, 29, 16, 24])
    with np.errstate(over="ignore"):
        ks = [u32(k1), u32(k2), u32(u32(k1) ^ u32(k2) ^ u32(0x1BD11BDA))]
        x0 = (x0 + ks[0]).astype(u32)
        x1 = (x1 + ks[1]).astype(u32)

        def rnd(x0, x1, r):
            x0 = (x0 + x1).astype(u32)
            x1 = ((x1 << u32(r)) | (x1 >> u32(32 - r))).astype(u32)
            return x0, (x1 ^ x0).astype(u32)

        for i, (a, b) in enumerate(((1, 2), (2, 0), (0, 1), (1, 2), (2, 0))):
            for r in rots[i % 2]:
                x0, x1 = rnd(x0, x1, r)
            x0 = (x0 + ks[a]).astype(u32)
            x1 = (x1 + ks[b] + u32(i + 1)).astype(u32)
        return x0, x1


def _subkey_chain(seed, n):
    k1, k2 = (seed >> 32) & 0xFFFFFFFF, seed & 0xFFFFFFFF
    subs = []
    for _ in range(n):
        b1, b2 = _threefry2x32_host(
            k1, k2, np.zeros(2, np.uint32), np.arange(2, dtype=np.uint32))
        k1, k2 = int(b1[0]), int(b2[0])
        subs.append((int(b1[1]), int(b2[1])))
    return np.array(subs, dtype=np.uint32)


_SUBKEYS = _subkey_chain(42, NSTEP)


def _tf_bits(k1, k2, x1):
    u32c = lambda v: jnp.uint32(v)
    ks0, ks1 = k1, k2
    ks2 = ks0 ^ ks1 ^ u32c(0x1BD11BDA)
    ks = (ks0, ks1, ks2)
    rots = ((13, 15, 26, 6), (17, 29, 16, 24))
    x0 = ks0
    x1 = x1 + ks1

    def rnd(x0, x1, r):
        x0 = x0 + x1
        x1 = (x1 << u32c(r)) | (x1 >> u32c(32 - r))
        return x0, x1 ^ x0

    for i, (a, b) in enumerate(((1, 2), (2, 0), (0, 1), (1, 2), (2, 0))):
        for r in rots[i % 2]:
            x0, x1 = rnd(x0, x1, r)
        x0 = x0 + ks[a]
        x1 = x1 + ks[b] + u32c(i + 1)
    return x0 ^ x1


def _sample_kernel(subkeys_ref, l_ref, tok_ref, prob_ref, rowmax_ref, rowsum_ref):
    s = pl.program_id(0)
    k1 = subkeys_ref[s, 0]
    k2 = subkeys_ref[s, 1]

    @pl.when(s == 0)
    def _softmax_stats():
        def stats_body(j, carry):
            m, acc = carry
            l = l_ref[:, pl.ds(j * W, W)]
            cm = jnp.max(l, axis=1, keepdims=True)
            m2 = jnp.maximum(m, cm)
            acc = acc * jnp.exp(m - m2) + jnp.sum(
                jnp.exp(l - m2), axis=1, keepdims=True)
            return m2, acc

        m0 = jnp.full((B, 1), NEG_INF, jnp.float32)
        a0 = jnp.zeros((B, 1), jnp.float32)
        m, acc = jax.lax.fori_loop(0, NCHUNK, stats_body, (m0, a0))
        rowmax_ref[...] = m
        rowsum_ref[...] = acc

    def step_body(j, carry):
        bv, bc, bl = carry
        l = l_ref[:, pl.ds(j * W, W)]
        colg = jax.lax.broadcasted_iota(jnp.int32, (B, W), 1) + j * W
        row = jax.lax.broadcasted_iota(jnp.int32, (B, W), 0)
        cnt = (row * V + colg).astype(jnp.uint32)
        bits = _tf_bits(k1, k2, cnt)
        fb = (bits >> jnp.uint32(9)) | jnp.uint32(0x3F800000)
        f = jax.lax.bitcast_convert_type(fb, jnp.float32) - jnp.float32(1.0)
        u = jnp.maximum(TINY, f * UMUL + TINY)
        g = -jnp.log(-jnp.log(u))
        val = g + l
        bm = jnp.max(val, axis=1, keepdims=True)
        cand = jnp.min(jnp.where(val == bm, colg, BIG_COL), axis=1, keepdims=True)
        lat = jnp.max(jnp.where(colg == cand, l, NEG_INF), axis=1, keepdims=True)
        better = bm > bv
        return (jnp.where(better, bm, bv),
                jnp.where(better, cand, bc),
                jnp.where(better, lat, bl))

    bv0 = jnp.full((B, 1), NEG_INF, jnp.float32)
    bc0 = jnp.zeros((B, 1), jnp.int32)
    bl0 = jnp.full((B, 1), NEG_INF, jnp.float32)
    _, bc, bl = jax.lax.fori_loop(0, NCHUNK, step_body, (bv0, bc0, bl0))
    tok_ref[...] = bc.reshape(1, 1, B)
    prob_ref[...] = (jnp.exp(bl - rowmax_ref[...]) / rowsum_ref[...]).reshape(1, 1, B)


def _run_sampling(last_logits):
    lpad = jnp.pad(last_logits, ((0, 0), (0, VPAD - V)),
                   constant_values=NEG_INF)
    grid_spec = pltpu.PrefetchScalarGridSpec(
        num_scalar_prefetch=1,
        grid=(NSTEP,),
        in_specs=[pl.BlockSpec((B, VPAD), lambda s, *_: (0, 0))],
        out_specs=[pl.BlockSpec((1, 1, B), lambda s, *_: (s, 0, 0)),
                   pl.BlockSpec((1, 1, B), lambda s, *_: (s, 0, 0))],
        scratch_shapes=[pltpu.VMEM((B, 1), jnp.float32),
                        pltpu.VMEM((B, 1), jnp.float32)],
    )
    tokens_sb, probs_sb = pl.pallas_call(
        _sample_kernel,
        grid_spec=grid_spec,
        out_shape=[jax.ShapeDtypeStruct((NSTEP, 1, B), jnp.int32),
                   jax.ShapeDtypeStruct((NSTEP, 1, B), jnp.float32)],
    )(jnp.asarray(_SUBKEYS), lpad)
    return tokens_sb[:, 0, :].T, probs_sb[:, 0, :].T


def kernel(hidden_states, logits, verifier_logits, W_head, b_head):
    last_logits = logits[:, -1, :]
    tokens_bs, probs_bs = _run_sampling(last_logits)
    draft_tokens = tokens_bs.reshape(B, KDRAFT, LDRAFT)
    draft_probs = probs_bs.reshape(B, KDRAFT, LDRAFT)
    accepted_mask = draft_probs >= jnp.float32(0.8)
    acceptance_ratio = jnp.mean(accepted_mask.astype(jnp.float32), axis=-1)
    best_draft_idx = jnp.argmax(acceptance_ratio, axis=-1)
    return (draft_tokens, draft_probs, accepted_mask, acceptance_ratio,
            best_draft_idx)
